# baseline (speedup 1.0000x reference)
import functools

import jax
import jax.numpy as jnp
from jax import lax
from jax.experimental import pallas as pl
from jax.experimental.pallas import tpu as pltpu
from jax.experimental.pallas import tpu_sc as plsc

_NUM_CORES = 2
_NUM_SUBCORES = 16
_NUM_WORKERS = _NUM_CORES * _NUM_SUBCORES

_ROWS_PER_CHUNK = 32


@functools.lru_cache(maxsize=None)
def _make_gather(batch, n_fields, dim):
    rows_per_w = batch // _NUM_WORKERS
    n_chunks = rows_per_w // _ROWS_PER_CHUNK
    mesh = plsc.VectorSubcoreMesh(core_axis_name="c", subcore_axis_name="s")

    @functools.partial(
        pl.kernel,
        mesh=mesh,
        out_type=jax.ShapeDtypeStruct((batch, n_fields, dim), jnp.float32),
        scratch_types=[
            pltpu.VMEM((rows_per_w, n_fields), jnp.int32),
            pltpu.VMEM((2, _ROWS_PER_CHUNK, n_fields, dim), jnp.float32),
            pltpu.SemaphoreType.DMA,
            pltpu.SemaphoreType.DMA,
        ],
        compiler_params=pltpu.CompilerParams(use_tc_tiling_on_sc=False),
    )
    def gather_kernel(x_hbm, table_hbm, out_hbm, idx_v, rows_v, g_sem, s_sem):
        wid = lax.axis_index("s") * _NUM_CORES + lax.axis_index("c")
        base = wid * rows_per_w

        pltpu.sync_copy(x_hbm.at[pl.ds(base, rows_per_w)], idx_v)

        def gather(i):
            buf = i % 2
            descs = []
            for r in range(_ROWS_PER_CHUNK):
                row = i * _ROWS_PER_CHUNK + r
                descs.append(
                    pltpu.async_copy(
                        table_hbm.at[idx_v.at[row]],
                        rows_v.at[buf, r],
                        g_sem,
                    )
                )
            return descs

        def store(i):
            return pltpu.async_copy(
                rows_v.at[i % 2],
                out_hbm.at[pl.ds(base + i * _ROWS_PER_CHUNK, _ROWS_PER_CHUNK)],
                s_sem,
            )

        gathers = [None] * n_chunks
        stores = [None] * n_chunks
        gathers[0] = gather(0)
        for i in range(n_chunks):
            if i + 1 < n_chunks:
                if i >= 1:
                    stores[i - 1].wait()
                gathers[i + 1] = gather(i + 1)
            for d in gathers[i]:
                d.wait()
            stores[i] = store(i)
        stores[n_chunks - 2].wait()
        stores[n_chunks - 1].wait()

    return gather_kernel


def kernel(x, weight):
    batch, n_fields = x.shape
    _, dim = weight.shape
    return _make_gather(batch, n_fields, dim)(x.astype(jnp.int32), weight)


# speedup vs baseline: 1.2267x; 1.2267x over previous
import functools

import jax
import jax.numpy as jnp
from jax import lax
from jax.experimental import pallas as pl
from jax.experimental.pallas import tpu as pltpu
from jax.experimental.pallas import tpu_sc as plsc

_NUM_CORES = 2
_NUM_SUBCORES = 16
_NUM_WORKERS = _NUM_CORES * _NUM_SUBCORES

_ROWS_PER_CHUNK = 32
_F_PAD = 32
_D_PAD = 128


@functools.lru_cache(maxsize=None)
def _make_gather(batch, n_fields, dim):
    rows_per_w = batch // _NUM_WORKERS
    n_chunks = rows_per_w // _ROWS_PER_CHUNK
    mesh = plsc.VectorSubcoreMesh(core_axis_name="c", subcore_axis_name="s")

    @functools.partial(
        pl.kernel,
        mesh=mesh,
        out_type=jax.ShapeDtypeStruct((batch, _F_PAD, _D_PAD), jnp.float32),
        scratch_types=[
            pltpu.VMEM((rows_per_w, n_fields), jnp.int32),
            pltpu.VMEM((2, _ROWS_PER_CHUNK, n_fields, dim), jnp.float32),
            pltpu.SemaphoreType.DMA,
            pltpu.SemaphoreType.DMA,
        ],
        compiler_params=pltpu.CompilerParams(use_tc_tiling_on_sc=False),
    )
    def gather_kernel(x_hbm, table_hbm, out_hbm, idx_v, rows_v, g_sem, s_sem):
        wid = lax.axis_index("s") * _NUM_CORES + lax.axis_index("c")
        base = wid * rows_per_w

        pltpu.sync_copy(x_hbm.at[pl.ds(base, rows_per_w)], idx_v)

        def gather(i):
            buf = i % 2
            descs = []
            for r in range(_ROWS_PER_CHUNK):
                row = i * _ROWS_PER_CHUNK + r
                descs.append(
                    pltpu.async_copy(
                        table_hbm.at[idx_v.at[row]],
                        rows_v.at[buf, r],
                        g_sem,
                    )
                )
            return descs

        def store(i):
            return pltpu.async_copy(
                rows_v.at[i % 2],
                out_hbm.at[
                    pl.ds(base + i * _ROWS_PER_CHUNK, _ROWS_PER_CHUNK),
                    pl.ds(0, n_fields),
                    pl.ds(0, dim),
                ],
                s_sem,
            )

        gathers = [None] * n_chunks
        stores = [None] * n_chunks
        gathers[0] = gather(0)
        for i in range(n_chunks):
            if i + 1 < n_chunks:
                if i >= 1:
                    stores[i - 1].wait()
                gathers[i + 1] = gather(i + 1)
            for d in gathers[i]:
                d.wait()
            stores[i] = store(i)
        stores[n_chunks - 2].wait()
        stores[n_chunks - 1].wait()

    return gather_kernel


def kernel(x, weight):
    batch, n_fields = x.shape
    _, dim = weight.shape
    out_big = _make_gather(batch, n_fields, dim)(x.astype(jnp.int32), weight)
    return out_big[:, :n_fields, :dim]
